# baseline (speedup 1.0000x reference)
import functools

import jax
import jax.numpy as jnp
from jax.experimental import pallas as pl
from jax.experimental.pallas import tpu as pltpu

B = 4096
STATE_DIM = 128
ACTION_DIM = 16
K_SHARED = 8192
K_PRIVATE = 2048
TOPK = 8

BLOCK_B = 128
NBLK = B // BLOCK_B


def _topk_threshold(logits, k, nslices=8):
    def _topm(x, nslices, m, absolute=False):
        w = x.shape[-1] // nslices
        slices = [x[:, i * w:(i + 1) * w] for i in range(nslices)]
        if absolute:
            slices = [jnp.abs(s) for s in slices]
        tops = []
        cur = slices
        for _ in range(m):
            s = functools.reduce(jnp.maximum, cur)
            tops.append(s)
            cur = [jnp.where(c == s, -1.0, c) for c in cur]
        return jnp.concatenate(tops, axis=-1)

    cand = _topm(logits, nslices, 2, absolute=True)
    if cand.shape[-1] >= 2048:
        cand = _topm(cand, nslices, 3)
    t = jnp.max(cand, axis=-1, keepdims=True)
    for _ in range(k - 1):
        masked = jnp.where(cand < t, cand, -1.0)
        t = jnp.max(masked, axis=-1, keepdims=True)
    return t


def _exact_tie_repair(absx):
    big = jnp.int32(absx.shape[-1])
    iota = jax.lax.broadcasted_iota(jnp.int32, absx.shape, 1)
    sel = jnp.zeros(absx.shape, dtype=jnp.bool_)
    for _ in range(TOPK):
        m = jnp.max(jnp.where(sel, -1.0, absx), axis=-1, keepdims=True)
        is_m = (~sel) & (absx == m)
        mi = jnp.min(jnp.where(is_m, iota, big), axis=-1, keepdims=True)
        sel = sel | (is_m & (iota == mi))
    return sel


def _fused_kernel(state_ref, action_ref, state_prev_ref,
                  shared_dict_ref, private_dict_ref,
                  W1_ref, b1_ref, W2_ref, b2_ref,
                  Wsh_ref, bsh_ref, Wa1_ref, ba1_ref, Wa2_ref, ba2_ref,
                  next_state_ref, alpha_ref,
                  ls_scr, h_scr):
    f32 = jnp.float32
    j = pl.program_id(0)
    par = jax.lax.rem(j, 2)
    prev = jax.lax.rem(j + 1, 2)

    def _mask_and_count(logits):
        t = _topk_threshold(logits, TOPK)
        ge = (logits >= t) | (logits <= -t)
        alpha = jnp.where(ge, logits, 0.0)
        cnt = jnp.max(jnp.sum(ge.astype(f32), axis=-1))
        return alpha, cnt

    def _private_logits(hh):
        aa = jnp.maximum(
            jax.lax.dot_general(hh, Wa1_ref[...], (((1,), (0,)), ((), ())),
                                preferred_element_type=f32) + ba1_ref[...], 0.0)
        return jax.lax.dot_general(aa, Wa2_ref[...], (((1,), (0,)), ((), ())),
                                   preferred_element_type=f32) + ba2_ref[...]

    logits_s = ls_scr[prev]
    logits_p = _private_logits(h_scr[prev])
    alpha_s, cnt_s = _mask_and_count(logits_s)
    alpha_ref[:, :K_SHARED] = alpha_s
    delta = jax.lax.dot_general(alpha_ref[:, :K_SHARED], shared_dict_ref[...],
                                (((1,), (1,)), ((), ())),
                                preferred_element_type=f32)
    alpha_p, cnt_p = _mask_and_count(logits_p)
    alpha_ref[:, K_SHARED:] = alpha_p
    delta += jax.lax.dot_general(alpha_ref[:, K_SHARED:], private_dict_ref[...],
                                 (((1,), (1,)), ((), ())),
                                 preferred_element_type=f32)
    next_state_ref[...] = state_prev_ref[...] + delta

    @pl.when(cnt_s > TOPK + 0.5)
    def _repair_shared():
        l = ls_scr[prev]
        alpha_ref[:, :K_SHARED] = jnp.where(
            _exact_tie_repair(jnp.abs(l)), l, 0.0)

    @pl.when(cnt_p > TOPK + 0.5)
    def _repair_private():
        l = _private_logits(h_scr[prev])
        alpha_ref[:, K_SHARED:] = jnp.where(
            _exact_tie_repair(jnp.abs(l)), l, 0.0)

    @pl.when((cnt_s > TOPK + 0.5) | (cnt_p > TOPK + 0.5))
    def _redecode():
        d = jax.lax.dot_general(alpha_ref[:, :K_SHARED], shared_dict_ref[...],
                                (((1,), (1,)), ((), ())),
                                preferred_element_type=f32)
        d += jax.lax.dot_general(alpha_ref[:, K_SHARED:], private_dict_ref[...],
                                 (((1,), (1,)), ((), ())),
                                 preferred_element_type=f32)
        next_state_ref[...] = state_prev_ref[...] + d

    x = jnp.concatenate([state_ref[...], action_ref[...]], axis=-1)
    h = jax.lax.dot_general(x, W1_ref[...],
                            (((1,), (0,)), ((), ())),
                            preferred_element_type=f32) + b1_ref[...]
    h = jnp.maximum(h, 0.0)
    h = jnp.maximum(
        jax.lax.dot_general(h, W2_ref[...], (((1,), (0,)), ((), ())),
                            preferred_element_type=f32) + b2_ref[...], 0.0)
    h_scr[par] = h
    ls_scr[par] = jax.lax.dot_general(
        h, Wsh_ref[...], (((1,), (0,)), ((), ())),
        preferred_element_type=f32) + bsh_ref[...]


@jax.jit
def kernel(state, action, shared_dict, private_dict,
           W1, b1, W2, b2, Wsh, bsh, Wa1, ba1, Wa2, ba2):
    grid = (NBLK + 1,)

    def cur_block(j):
        return (jnp.minimum(j, NBLK - 1), 0)

    def prev_block(j):
        return (jnp.maximum(j - 1, 0), 0)

    def whole(*_):
        return (0, 0)

    def whole1(*_):
        return (0,)

    in_specs = [
        pl.BlockSpec((BLOCK_B, STATE_DIM), cur_block),
        pl.BlockSpec((BLOCK_B, ACTION_DIM), cur_block),
        pl.BlockSpec((BLOCK_B, STATE_DIM), prev_block),
        pl.BlockSpec((STATE_DIM, K_SHARED), whole),
        pl.BlockSpec((STATE_DIM, K_PRIVATE), whole),
        pl.BlockSpec(W1.shape, whole),
        pl.BlockSpec(b1.shape, whole1),
        pl.BlockSpec(W2.shape, whole),
        pl.BlockSpec(b2.shape, whole1),
        pl.BlockSpec(Wsh.shape, whole),
        pl.BlockSpec(bsh.shape, whole1),
        pl.BlockSpec(Wa1.shape, whole),
        pl.BlockSpec(ba1.shape, whole1),
        pl.BlockSpec(Wa2.shape, whole),
        pl.BlockSpec(ba2.shape, whole1),
    ]
    out_specs = [
        pl.BlockSpec((BLOCK_B, STATE_DIM), prev_block),
        pl.BlockSpec((BLOCK_B, K_SHARED + K_PRIVATE), prev_block),
    ]
    out_shapes = [
        jax.ShapeDtypeStruct((B, STATE_DIM), jnp.float32),
        jax.ShapeDtypeStruct((B, K_SHARED + K_PRIVATE), jnp.float32),
    ]
    next_state, alpha = pl.pallas_call(
        _fused_kernel,
        grid=grid,
        in_specs=in_specs,
        out_specs=out_specs,
        out_shape=out_shapes,
        scratch_shapes=[
            pltpu.VMEM((2, BLOCK_B, K_SHARED), jnp.float32),
            pltpu.VMEM((2, BLOCK_B, 256), jnp.float32),
        ],
    )(state, action, state, shared_dict, private_dict,
      W1, b1, W2, b2, Wsh, bsh, Wa1, ba1, Wa2, ba2)
    return next_state, alpha


# speedup vs baseline: 1.1652x; 1.1652x over previous
import functools

import jax
import jax.numpy as jnp
from jax.experimental import pallas as pl
from jax.experimental.pallas import tpu as pltpu

B = 4096
STATE_DIM = 128
ACTION_DIM = 16
K_SHARED = 8192
K_PRIVATE = 2048
TOPK = 8

BLOCK_B = 128


def _topk_threshold(absx, k, nslices=8):
    def _top2_tournament(x, nslices):
        w = x.shape[-1] // nslices
        pairs = [(x[:, i * w:(i + 1) * w], None) for i in range(nslices)]

        def merge(p, q):
            (m1, s1), (m2, s2) = p, q
            lo = jnp.minimum(m1, m2)
            if s1 is None and s2 is None:
                return jnp.maximum(m1, m2), lo
            loser_side = jnp.where(m1 >= m2, s1, s2)
            return jnp.maximum(m1, m2), jnp.maximum(lo, loser_side)

        while len(pairs) > 1:
            pairs = [merge(pairs[i], pairs[i + 1])
                     for i in range(0, len(pairs), 2)]
        return jnp.concatenate(pairs[0], axis=-1)

    def _topm(x, nslices, m):
        w = x.shape[-1] // nslices
        slices = [x[:, i * w:(i + 1) * w] for i in range(nslices)]
        tops = []
        cur = slices
        for _ in range(m):
            s = functools.reduce(jnp.maximum, cur)
            tops.append(s)
            cur = [jnp.where(c == s, -1.0, c) for c in cur]
        return jnp.concatenate(tops, axis=-1)

    cand = _top2_tournament(absx, nslices)
    if cand.shape[-1] >= 2048:
        cand = _topm(cand, nslices, 3)
    t = jnp.max(cand, axis=-1, keepdims=True)
    for _ in range(k - 1):
        masked = jnp.where(cand < t, cand, -1.0)
        t = jnp.max(masked, axis=-1, keepdims=True)
    return t


def _exact_tie_repair(absx):
    big = jnp.int32(absx.shape[-1])
    iota = jax.lax.broadcasted_iota(jnp.int32, absx.shape, 1)
    sel = jnp.zeros(absx.shape, dtype=jnp.bool_)
    avail = absx
    for _ in range(TOPK):
        m = jnp.max(avail, axis=-1, keepdims=True)
        is_m = avail == m
        mi = jnp.min(jnp.where(is_m, iota, big), axis=-1, keepdims=True)
        pick = is_m & (iota == mi)
        sel = sel | pick
        avail = jnp.where(pick, -1.0, avail)
    return sel


def _fused_kernel(state_ref, action_ref,
                  shared_dict_ref, private_dict_ref,
                  W1_ref, b1_ref, W2_ref, b2_ref,
                  Wsh_ref, bsh_ref, Wa1_ref, ba1_ref, Wa2_ref, ba2_ref,
                  next_state_ref, alpha_ref):
    f32 = jnp.float32
    state = state_ref[...]
    action = action_ref[...]

    x = jnp.concatenate([state, action], axis=-1)
    h = jax.lax.dot_general(x, W1_ref[...],
                            (((1,), (0,)), ((), ())),
                            preferred_element_type=f32) + b1_ref[...]
    h = jnp.maximum(h, 0.0)
    h = jnp.maximum(
        jax.lax.dot_general(h, W2_ref[...], (((1,), (0,)), ((), ())),
                            preferred_element_type=f32) + b2_ref[...], 0.0)

    def _shared_logits():
        return jax.lax.dot_general(h, Wsh_ref[...], (((1,), (0,)), ((), ())),
                                   preferred_element_type=f32) + bsh_ref[...]

    def _private_logits():
        a = jnp.maximum(
            jax.lax.dot_general(h, Wa1_ref[...], (((1,), (0,)), ((), ())),
                                preferred_element_type=f32) + ba1_ref[...], 0.0)
        return jax.lax.dot_general(a, Wa2_ref[...], (((1,), (0,)), ((), ())),
                                   preferred_element_type=f32) + ba2_ref[...]

    def _mask_and_count(logits):
        absx = jnp.abs(logits)
        t = _topk_threshold(absx, TOPK)
        ge = absx >= t
        alpha = jnp.where(ge, logits, 0.0)
        cnt = jnp.max(jnp.sum(ge.astype(f32), axis=-1))
        return alpha, cnt

    alpha_s, cnt_s = _mask_and_count(_shared_logits())
    alpha_ref[:, :K_SHARED] = alpha_s
    delta = jax.lax.dot_general(alpha_s, shared_dict_ref[...],
                                (((1,), (1,)), ((), ())),
                                preferred_element_type=f32)
    alpha_p, cnt_p = _mask_and_count(_private_logits())
    alpha_ref[:, K_SHARED:] = alpha_p
    delta += jax.lax.dot_general(alpha_p, private_dict_ref[...],
                                 (((1,), (1,)), ((), ())),
                                 preferred_element_type=f32)
    next_state_ref[...] = state + delta

    @pl.when(cnt_s > TOPK + 0.5)
    def _repair_shared():
        logits = _shared_logits()
        alpha_ref[:, :K_SHARED] = jnp.where(
            _exact_tie_repair(jnp.abs(logits)), logits, 0.0)

    @pl.when(cnt_p > TOPK + 0.5)
    def _repair_private():
        logits = _private_logits()
        alpha_ref[:, K_SHARED:] = jnp.where(
            _exact_tie_repair(jnp.abs(logits)), logits, 0.0)

    @pl.when((cnt_s > TOPK + 0.5) | (cnt_p > TOPK + 0.5))
    def _redecode():
        d = jax.lax.dot_general(alpha_ref[:, :K_SHARED], shared_dict_ref[...],
                                (((1,), (1,)), ((), ())),
                                preferred_element_type=f32)
        d += jax.lax.dot_general(alpha_ref[:, K_SHARED:], private_dict_ref[...],
                                 (((1,), (1,)), ((), ())),
                                 preferred_element_type=f32)
        next_state_ref[...] = state + d


@jax.jit
def kernel(state, action, shared_dict, private_dict,
           W1, b1, W2, b2, Wsh, bsh, Wa1, ba1, Wa2, ba2):
    grid = (B // BLOCK_B,)

    def row_block(i):
        return (i, 0)

    def whole(*_):
        return (0, 0)

    def whole1(*_):
        return (0,)

    in_specs = [
        pl.BlockSpec((BLOCK_B, STATE_DIM), row_block),
        pl.BlockSpec((BLOCK_B, ACTION_DIM), row_block),
        pl.BlockSpec((STATE_DIM, K_SHARED), whole),
        pl.BlockSpec((STATE_DIM, K_PRIVATE), whole),
        pl.BlockSpec(W1.shape, whole),
        pl.BlockSpec(b1.shape, whole1),
        pl.BlockSpec(W2.shape, whole),
        pl.BlockSpec(b2.shape, whole1),
        pl.BlockSpec(Wsh.shape, whole),
        pl.BlockSpec(bsh.shape, whole1),
        pl.BlockSpec(Wa1.shape, whole),
        pl.BlockSpec(ba1.shape, whole1),
        pl.BlockSpec(Wa2.shape, whole),
        pl.BlockSpec(ba2.shape, whole1),
    ]
    out_specs = [
        pl.BlockSpec((BLOCK_B, STATE_DIM), row_block),
        pl.BlockSpec((BLOCK_B, K_SHARED + K_PRIVATE), row_block),
    ]
    out_shapes = [
        jax.ShapeDtypeStruct((B, STATE_DIM), jnp.float32),
        jax.ShapeDtypeStruct((B, K_SHARED + K_PRIVATE), jnp.float32),
    ]
    next_state, alpha = pl.pallas_call(
        _fused_kernel,
        grid=grid,
        in_specs=in_specs,
        out_specs=out_specs,
        out_shape=out_shapes,
        compiler_params=pltpu.CompilerParams(
            vmem_limit_bytes=112 * 1024 * 1024),
    )(state, action, shared_dict, private_dict,
      W1, b1, W2, b2, Wsh, bsh, Wa1, ba1, Wa2, ba2)
    return next_state, alpha
